# baseline (speedup 1.0000x reference)
import jax
import jax.numpy as jnp
from jax.experimental import pallas as pl
from jax.experimental.pallas import tpu as pltpu

_K = 1024
_CHUNK_ROWS = 64
_CHUNK = _CHUNK_ROWS * 128
_NEG = float("-inf")


def _flat_iota(shape):
    r = jax.lax.broadcasted_iota(jnp.int32, shape, 0)
    l = jax.lax.broadcasted_iota(jnp.int32, shape, 1)
    return r * 128 + l


def _ce(x, fi, d, asc):
    if d < 128:
        left = pltpu.roll(x, 128 - d, axis=1)
        right = pltpu.roll(x, d, axis=1)
    else:
        dr = d // 128
        rows = x.shape[0]
        left = pltpu.roll(x, rows - dr, axis=0)
        right = pltpu.roll(x, dr, axis=0)
    lower = (fi & d) == 0
    p = jnp.where(lower, left, right)
    return jnp.where(asc == lower, jnp.minimum(x, p), jnp.maximum(x, p))


def _chunk_topk(x, fi):
    k = 2
    while k <= _K:
        asc = (fi & k) == 0
        d = k // 2
        while d >= 1:
            x = _ce(x, fi, d, asc)
            d //= 2
        k *= 2
    while x.shape[0] > 8:
        rows = x.shape[0]
        y = jnp.maximum(x, pltpu.roll(x, rows - 8, axis=0))
        x = jnp.concatenate([y[i:i + 8] for i in range(0, rows, 16)], axis=0)
        fi2 = _flat_iota(x.shape)
        asc = (fi2 & _K) == 0
        d = _K // 2
        while d >= 1:
            x = _ce(x, fi2, d, asc)
            d //= 2
    return x


def _make_body(n_valid, grid):
    def _body(sub_ref, x_ref, out_ref, b_ref):
        pi = pl.program_id(0)

        @pl.when(pi == 0)
        def _():
            b_ref[...] = jnp.full((8, 128), _NEG, jnp.float32)

        v = x_ref[...]
        diff = pltpu.roll(v, 127, axis=1) - v
        sub = sub_ref[0] != 0
        l = jnp.where(sub, diff, -diff)
        l = jnp.where(jnp.isinf(l), 0.0, l)
        l = jnp.where(jnp.isnan(l), 0.0, l)
        fi = _flat_iota(v.shape)
        valid = ((fi & 1) == 0) & (pi * _CHUNK + fi < n_valid)
        x = jnp.where(valid, l, _NEG)

        top = _chunk_topk(x, fi)

        b = jnp.maximum(b_ref[...], top)
        fi3 = _flat_iota(b.shape)
        asc = jnp.zeros(b.shape, jnp.bool_)
        d = _K // 2
        while d >= 1:
            b = _ce(b, fi3, d, asc)
            d //= 2
        b_ref[...] = b

        @pl.when(pi == grid - 1)
        def _():
            out_ref[...] = b

    return _body


def _topk_flat(flat, sub, interpret=False):
    n = flat.shape[0]
    grid = (n + _CHUNK - 1) // _CHUNK
    padded = grid * _CHUNK
    if padded != n:
        flat = jnp.concatenate([flat, jnp.zeros((padded - n,), flat.dtype)])
    x = flat.reshape(grid * _CHUNK_ROWS, 128)
    out = pl.pallas_call(
        _make_body(n, grid),
        grid=(grid,),
        in_specs=[
            pl.BlockSpec(memory_space=pltpu.SMEM),
            pl.BlockSpec((_CHUNK_ROWS, 128), lambda i: (i, 0)),
        ],
        out_specs=pl.BlockSpec((8, 128), lambda i: (0, 0)),
        out_shape=jax.ShapeDtypeStruct((8, 128), jnp.float32),
        scratch_shapes=[pltpu.VMEM((8, 128), jnp.float32)],
        interpret=interpret,
    )(sub, x)
    return out.reshape(_K)


def kernel(dgm, issublevel):
    flat = dgm[0].reshape(-1)
    sub = jnp.asarray(issublevel, jnp.int32).reshape(1)
    return _topk_flat(flat, sub)


# speedup vs baseline: 1.6828x; 1.6828x over previous
import jax
import jax.numpy as jnp
from jax.experimental import pallas as pl
from jax.experimental.pallas import tpu as pltpu

_K = 1024
_CHUNK_ROWS = 1024
_CHUNK = _CHUNK_ROWS * 128
_NEG = float("-inf")


def _flat_iota(shape):
    r = jax.lax.broadcasted_iota(jnp.int32, shape, 0)
    l = jax.lax.broadcasted_iota(jnp.int32, shape, 1)
    return r * 128 + l


def _ce(x, fi, d, asc):
    if d < 128:
        left = pltpu.roll(x, 128 - d, axis=1)
        right = pltpu.roll(x, d, axis=1)
    else:
        dr = d // 128
        rows = x.shape[0]
        left = pltpu.roll(x, rows - dr, axis=0)
        right = pltpu.roll(x, dr, axis=0)
    lower = (fi & d) == 0
    p = jnp.where(lower, left, right)
    return jnp.where(asc == lower, jnp.minimum(x, p), jnp.maximum(x, p))


def _chunk_topk(x, fi):
    k = 2
    while k <= _K:
        asc = (fi & k) == 0
        d = k // 2
        while d >= 1:
            x = _ce(x, fi, d, asc)
            d //= 2
        k *= 2
    while x.shape[0] > 8:
        rows = x.shape[0]
        y = jnp.maximum(x, pltpu.roll(x, rows - 8, axis=0))
        x = jnp.concatenate([y[i:i + 8] for i in range(0, rows, 16)], axis=0)
        fi2 = _flat_iota(x.shape)
        asc = (fi2 & _K) == 0
        d = _K // 2
        while d >= 1:
            x = _ce(x, fi2, d, asc)
            d //= 2
    return x


def _make_body(n_valid, grid):
    def _body(sub_ref, x_ref, out_ref, b_ref):
        pi = pl.program_id(0)

        @pl.when(pi == 0)
        def _():
            b_ref[...] = jnp.full((8, 128), _NEG, jnp.float32)

        v = x_ref[...]
        diff = pltpu.roll(v, 127, axis=1) - v
        sub = sub_ref[0] != 0
        l = jnp.where(sub, diff, -diff)
        l = jnp.where(jnp.isinf(l), 0.0, l)
        l = jnp.where(jnp.isnan(l), 0.0, l)
        fi = _flat_iota(v.shape)
        valid = ((fi & 1) == 0) & (pi * _CHUNK + fi < n_valid)
        x = jnp.where(valid, l, _NEG)

        top = _chunk_topk(x, fi)

        b = jnp.maximum(b_ref[...], top)
        fi3 = _flat_iota(b.shape)
        asc = jnp.zeros(b.shape, jnp.bool_)
        d = _K // 2
        while d >= 1:
            b = _ce(b, fi3, d, asc)
            d //= 2
        b_ref[...] = b

        @pl.when(pi == grid - 1)
        def _():
            out_ref[...] = b

    return _body


def _topk_flat(flat, sub, interpret=False):
    n = flat.shape[0]
    grid = (n + _CHUNK - 1) // _CHUNK
    padded = grid * _CHUNK
    if padded != n:
        flat = jnp.concatenate([flat, jnp.zeros((padded - n,), flat.dtype)])
    x = flat.reshape(grid * _CHUNK_ROWS, 128)
    out = pl.pallas_call(
        _make_body(n, grid),
        grid=(grid,),
        in_specs=[
            pl.BlockSpec(memory_space=pltpu.SMEM),
            pl.BlockSpec((_CHUNK_ROWS, 128), lambda i: (i, 0)),
        ],
        out_specs=pl.BlockSpec((8, 128), lambda i: (0, 0)),
        out_shape=jax.ShapeDtypeStruct((8, 128), jnp.float32),
        scratch_shapes=[pltpu.VMEM((8, 128), jnp.float32)],
        interpret=interpret,
    )(sub, x)
    return out.reshape(_K)


def kernel(dgm, issublevel):
    flat = dgm[0].reshape(-1)
    sub = jnp.asarray(issublevel, jnp.int32).reshape(1)
    return _topk_flat(flat, sub)
